# baseline (speedup 1.0000x reference)
import functools

import jax
import jax.numpy as jnp
from jax import lax
from jax.experimental import pallas as pl
from jax.experimental.pallas import tpu as pltpu
from jax.experimental.pallas import tpu_sc as plsc

D = 128
NUM_CRYSTALS = 1000
CP = 1024
NP_ATOMS = 10240
CLAMP = 60.0
NC, NS, L = 2, 16, 16
NW = NC * NS


def _full(a):
    return pl.BlockSpec(a.shape, lambda i: (0,) * a.ndim)


def _row2(x):
    return x.reshape(1, -1)


def _tc_embed(orig, emb):
    N = orig.shape[0]
    W, b = emb
    B = 2000

    def body(x, w, br, out):
        out[...] = jnp.dot(x[...], w[...], preferred_element_type=jnp.float32) + br[...]

    return pl.pallas_call(
        body,
        grid=(N // B,),
        in_specs=[pl.BlockSpec((B, D), lambda i: (i, 0)), _full(W), _full(_row2(b))],
        out_specs=pl.BlockSpec((B, D), lambda i: (i, 0)),
        out_shape=jax.ShapeDtypeStruct((N, D), jnp.float32),
    )(orig, W, _row2(b))


def _gate_coeff(w, gate):
    return w * jnp.exp(jnp.clip(gate, -CLAMP, CLAMP))


def _tc_edge_mlp(self_f, nbr_f, nbr_w, msg, pool, block=1280):
    M = self_f.shape[0]
    (W1, b1), (W2, b2) = msg
    (P1, q1), (P2, q2), (P3, q3) = pool
    W1a, W1b = W1[:D], W1[D:]

    bf = jnp.bfloat16

    def body(sf, nf, nw, w1a, w1b, b1r, w2, b2r, p1, q1r, p2, q2r, p3, q3r,
             scaled_o, c_o):
        h = jnp.dot(sf[...].astype(bf), w1a[...],
                    preferred_element_type=jnp.float32)
        h += jnp.dot(nf[...].astype(bf), w1b[...],
                     preferred_element_type=jnp.float32)
        h = jnp.maximum(h + b1r[...], 0.0)
        fea = jnp.dot(h.astype(bf), w2[...],
                      preferred_element_type=jnp.float32) + b2r[...]
        g = jnp.maximum(
            jnp.dot(fea.astype(bf), p1[...],
                    preferred_element_type=jnp.float32) + q1r[...], 0.0)
        g = jnp.maximum(
            jnp.dot(g.astype(bf), p2[...],
                    preferred_element_type=jnp.float32) + q2r[...], 0.0)
        gate = jnp.dot(g, p3[...], preferred_element_type=jnp.float32) + q3r[...]
        c = _gate_coeff(nw[...], gate)
        scaled_o[...] = c * fea
        c_o[...] = c

    args = (self_f, nbr_f, nbr_w, W1a.astype(bf), W1b.astype(bf), _row2(b1),
            W2.astype(bf), _row2(b2), P1.astype(bf), _row2(q1),
            P2.astype(bf), _row2(q2), P3, _row2(q3))
    eb = pl.BlockSpec((block, D), lambda i: (i, 0))
    wb = pl.BlockSpec((block, 1), lambda i: (i, 0))
    return pl.pallas_call(
        body,
        grid=(M // block,),
        in_specs=[eb, eb, wb] + [_full(a) for a in args[3:]],
        out_specs=[eb, wb],
        out_shape=[jax.ShapeDtypeStruct((M, D), jnp.float32),
                   jax.ShapeDtypeStruct((M, 1), jnp.float32)],
    )(*args)


def _tc_combine(fea_partials, den_t, table):
    N = table.shape[0]
    B = 2000

    def body(p, dn, t, out):
        s = p[0] + p[1]
        den = jnp.sum(dn[...], axis=1, keepdims=True)
        out[...] = t[...] + s / (den + 1e-13)

    return pl.pallas_call(
        body,
        grid=(N // B,),
        in_specs=[pl.BlockSpec((NC, B, D), lambda i: (0, i, 0)),
                  pl.BlockSpec((B, NW), lambda i: (i, 0)),
                  pl.BlockSpec((B, D), lambda i: (i, 0))],
        out_specs=pl.BlockSpec((B, D), lambda i: (i, 0)),
        out_shape=jax.ShapeDtypeStruct((N, D), jnp.float32),
    )(fea_partials, den_t, table)


def _tc_cry_gate(table_p, weights_p, cry):
    NPad = table_p.shape[0]
    (C1, d1), (C2, d2), (C3, d3), (C4, d4) = cry
    B = 2048

    def body(t, w, c1, e1, c2, e2, c3, e3, c4, e4, scaled_o, c_o):
        x = t[...]
        h = jnp.maximum(jnp.dot(x, c1[...], preferred_element_type=jnp.float32) + e1[...], 0.0)
        h = jnp.maximum(jnp.dot(h, c2[...], preferred_element_type=jnp.float32) + e2[...], 0.0)
        h = jnp.maximum(jnp.dot(h, c3[...], preferred_element_type=jnp.float32) + e3[...], 0.0)
        gate = jnp.dot(h, c4[...], preferred_element_type=jnp.float32) + e4[...]
        c = _gate_coeff(w[...], gate)
        scaled_o[...] = c * x
        c_o[...] = c

    args = (table_p, weights_p, C1, _row2(d1), C2, _row2(d2), C3, _row2(d3),
            C4, _row2(d4))
    return pl.pallas_call(
        body,
        grid=(NPad // B,),
        in_specs=[pl.BlockSpec((B, D), lambda i: (i, 0)),
                  pl.BlockSpec((B, 1), lambda i: (i, 0))] +
                 [_full(a) for a in args[2:]],
        out_specs=[pl.BlockSpec((B, D), lambda i: (i, 0)),
                   pl.BlockSpec((B, 1), lambda i: (i, 0))],
        out_shape=[jax.ShapeDtypeStruct((NPad, D), jnp.float32),
                   jax.ShapeDtypeStruct((NPad, 1), jnp.float32)],
    )(*args)


def _tc_out(fea_partials, den_t, out_fcs, out_res, out_lin):
    Wo, bo = out_lin

    def body(p, dn, *refs):
        s = p[0] + p[1]
        den = jnp.sum(dn[...], axis=1, keepdims=True)
        fea = s / (den + 1e-13)
        args = refs[:-1]
        out = refs[-1]
        k = 0
        for _ in range(4):
            wf, bf, wr = args[k], args[k + 1], args[k + 2]
            k += 3
            fea = jnp.maximum(
                jnp.dot(fea, wf[...], preferred_element_type=jnp.float32) + bf[...], 0.0
            ) + jnp.dot(fea, wr[...], preferred_element_type=jnp.float32)
        out[...] = jnp.dot(fea, args[k][...], preferred_element_type=jnp.float32) + args[k + 1][...]

    flat = []
    for (Wf, bf), (Wr,) in zip(out_fcs, out_res):
        flat += [Wf, _row2(bf), Wr]
    flat += [Wo, _row2(bo)]
    return pl.pallas_call(
        body,
        grid=(1,),
        in_specs=[_full(fea_partials), _full(den_t)] + [_full(a) for a in flat],
        out_specs=pl.BlockSpec((CP, 2), lambda i: (0, 0)),
        out_shape=jax.ShapeDtypeStruct((CP, 2), jnp.float32),
    )(fea_partials, den_t, *flat)


def _sc_gather(table, weights, sidx3, nidx3, nb=4):
    N = table.shape[0]
    _, K, CH = sidx3.shape
    epw = K * CH
    M = NW * epw
    ko = K // nb
    mesh = plsc.VectorSubcoreMesh(core_axis_name="c", subcore_axis_name="s")

    scratch = ([pltpu.VMEM((K, CH), jnp.int32)] * 2 +
               [pltpu.VMEM((N,), jnp.float32)] +
               [pltpu.VMEM((CH, D), jnp.float32)] * (2 * nb) +
               [pltpu.VMEM((CH,), jnp.float32)] * nb +
               [pltpu.SemaphoreType.DMA] * (3 * nb))

    @functools.partial(
        pl.kernel,
        out_type=[jax.ShapeDtypeStruct((M, D), jnp.float32),
                  jax.ShapeDtypeStruct((M, D), jnp.float32),
                  jax.ShapeDtypeStruct((M,), jnp.float32)],
        mesh=mesh,
        scratch_types=scratch,
        compiler_params=pltpu.CompilerParams(needs_layout_passes=False),
    )
    def k(table_hbm, w_hbm, sidx_hbm, nidx_hbm, self_out, nbr_out, w_out,
          *scr):
        sidxv, nidxv, wtab = scr[0], scr[1], scr[2]
        srv = scr[3:3 + nb]
        nrv = scr[3 + nb:3 + 2 * nb]
        wv = scr[3 + 2 * nb:3 + 3 * nb]
        sg = scr[3 + 3 * nb:3 + 4 * nb]
        ng = scr[3 + 4 * nb:3 + 5 * nb]
        ws = scr[3 + 5 * nb:3 + 6 * nb]
        wid = lax.axis_index("s") * NC + lax.axis_index("c")
        pltpu.sync_copy(w_hbm, wtab)
        pltpu.sync_copy(sidx_hbm.at[wid], sidxv)
        pltpu.sync_copy(nidx_hbm.at[wid], nidxv)

        def chunk_gather(c, b):
            return (pltpu.async_copy(table_hbm.at[sidxv.at[c]], srv[b], sg[b]),
                    pltpu.async_copy(table_hbm.at[nidxv.at[c]], nrv[b], ng[b]))

        def chunk_drain(c, b, h1, h2):
            base = wid * epw + c * CH
            for j in range(CH // L):
                sl = pl.ds(j * L, L)
                wv[b][sl] = plsc.load_gather(wtab, [nidxv[c, sl]])
            h1.wait()
            h2.wait()
            return (pltpu.async_copy(srv[b], self_out.at[pl.ds(base, CH)], ws[b]),
                    pltpu.async_copy(nrv[b], nbr_out.at[pl.ds(base, CH)], ws[b]),
                    pltpu.async_copy(wv[b], w_out.at[pl.ds(base, CH)], ws[b]))

        def step(kk, carry):
            hs = [chunk_gather(kk * nb + b, b) for b in range(nb)]
            whs = []
            for b in range(nb):
                whs += chunk_drain(kk * nb + b, b, *hs[b])
            for h in whs:
                h.wait()
            return carry

        lax.fori_loop(0, ko, step, 0)
        whs = []
        hs = [chunk_gather(c, c - ko * nb) for c in range(ko * nb, K)]
        for c in range(ko * nb, K):
            whs += chunk_drain(c, c - ko * nb, *hs[c - ko * nb])
        for h in whs:
            h.wait()

    return k(table, weights, sidx3, nidx3)


def _sc_segment_sum(scaled, cvec, idx3, zeros_seg):
    _, K, CH = idx3.shape
    epw = K * CH
    nseg = zeros_seg.shape[0]
    mesh = plsc.VectorSubcoreMesh(core_axis_name="c", subcore_axis_name="s")

    nb = 1 if nseg * D > 600_000 else min(2, K)
    ko = K // nb
    rpt = nseg // NS - (nseg // NS) % 8
    tail_rows = nseg - rpt * NS
    scratch = ([pltpu.VMEM((nseg,), jnp.float32),
                pltpu.VMEM_SHARED((nseg, D), jnp.float32),
                pltpu.VMEM((K, CH), jnp.int32),
                pltpu.VMEM((epw,), jnp.float32)] +
               [pltpu.VMEM((CH, D), jnp.float32)] * nb +
               [pltpu.SemaphoreType.DMA] * (2 * nb))

    @functools.partial(
        pl.kernel,
        out_type=[jax.ShapeDtypeStruct((NC, nseg, D), jnp.float32),
                  jax.ShapeDtypeStruct((NW, nseg), jnp.float32)],
        mesh=mesh,
        scratch_types=scratch,
        compiler_params=pltpu.CompilerParams(needs_layout_passes=False),
    )
    def k(scaled_hbm, cvec_hbm, idx_hbm, zeros_hbm, fea_out, den_out, *scr):
        dacc, acc, idxv, cvecv = scr[0], scr[1], scr[2], scr[3]
        rv = scr[4:4 + nb]
        sl_sem = scr[4 + nb:4 + 2 * nb]
        ss_sem = scr[4 + 2 * nb:4 + 3 * nb]
        cid = lax.axis_index("c")
        sid = lax.axis_index("s")
        wid = sid * NC + cid

        @pl.when(sid == 0)
        def _init():
            pltpu.sync_copy(zeros_hbm, acc)

        pltpu.sync_copy(idx_hbm.at[wid], idxv)
        pltpu.sync_copy(cvec_hbm.at[pl.ds(wid * epw, epw)], cvecv)
        zeros16 = jnp.zeros((L,), jnp.float32)

        def zstep(j, carry):
            dacc[pl.ds(j * L, L)] = zeros16
            return carry

        lax.fori_loop(0, nseg // L, zstep, 0)
        plsc.subcore_barrier()

        def chunk_load(c, b):
            base = wid * epw + c * CH
            return pltpu.async_copy(scaled_hbm.at[pl.ds(base, CH)], rv[b],
                                    sl_sem[b])

        def chunk_add(c, b, h):
            h.wait()
            out = pltpu.async_copy(rv[b], acc.at[idxv.at[c]], ss_sem[b],
                                   add=True)
            for j in range(CH // L):
                plsc.addupdate_scatter(dacc, [idxv[c, pl.ds(j * L, L)]],
                                       cvecv[pl.ds(c * CH + j * L, L)])
            return out

        def step(kk, carry):
            hs = [chunk_load(kk * nb + b, b) for b in range(nb)]
            shs = [chunk_add(kk * nb + b, b, hs[b]) for b in range(nb)]
            for h in shs:
                h.wait()
            return carry

        lax.fori_loop(0, ko, step, 0)
        hs = [chunk_load(wid * 0 + c, c - ko * nb) for c in range(ko * nb, K)]
        shs = [chunk_add(wid * 0 + c, c - ko * nb, hs[c - ko * nb])
               for c in range(ko * nb, K)]
        for h in shs:
            h.wait()
        plsc.subcore_barrier()
        r0 = sid * rpt
        pltpu.sync_copy(acc.at[pl.ds(r0, rpt)],
                        fea_out.at[cid, pl.ds(r0, rpt)])
        if tail_rows:
            @pl.when(sid == 0)
            def _dump_tail():
                pltpu.sync_copy(
                    acc.at[pl.ds(rpt * NS, tail_rows)],
                    fea_out.at[cid, pl.ds(rpt * NS, tail_rows)])
        pltpu.sync_copy(dacc, den_out.at[wid])

    return k(scaled, cvec, idx3, zeros_seg)


def kernel(atom_weights, orig_atom_fea, params, self_fea_idx, nbr_fea_idx,
           crystal_atom_idx):
    aw = atom_weights.astype(jnp.float32)
    sidx = self_fea_idx.astype(jnp.int32)
    nidx = nbr_fea_idx.astype(jnp.int32)
    cidx = crystal_atom_idx.astype(jnp.int32)
    N = aw.shape[0]
    M = sidx.shape[0]
    aw1 = aw.reshape(N)

    che = 80
    ke = M // (NW * che)
    sidx3 = sidx.reshape(NW, ke, che)
    nidx3 = nidx.reshape(NW, ke, che)
    zerosN = jnp.zeros((N, D), jnp.float32)

    table = _tc_embed(orig_atom_fea, params["emb"])
    for g in params["graphs"]:
        self_f, nbr_f, nbr_w = _sc_gather(table, aw1, sidx3, nidx3)
        scaled, cvec = _tc_edge_mlp(self_f, nbr_f, nbr_w.reshape(M, 1),
                                    g["msg"], g["pool"])
        fea_p, den_p = _sc_segment_sum(scaled, cvec.reshape(M), sidx3, zerosN)
        table = _tc_combine(fea_p, den_p.T, table)

    table_p = jnp.pad(table, ((0, NP_ATOMS - N), (0, 0)))
    aw_p = jnp.pad(aw, ((0, NP_ATOMS - N), (0, 0)))
    scaled_c, cvec_c = _tc_cry_gate(table_p, aw_p, params["cry"])
    chc = 64
    kc = NP_ATOMS // (NW * chc)
    cidx3 = jnp.pad(cidx, (0, NP_ATOMS - N)).reshape(NW, kc, chc)
    zerosC = jnp.zeros((CP, D), jnp.float32)
    fea_pc, den_pc = _sc_segment_sum(scaled_c, cvec_c.reshape(NP_ATOMS),
                                     cidx3, zerosC)
    out = _tc_out(fea_pc, den_pc.T, params["out_fcs"], params["out_res"],
                  params["out"])
    return out[:NUM_CRYSTALS]


# speedup vs baseline: 1.0585x; 1.0585x over previous
import functools

import jax
import jax.numpy as jnp
from jax import lax
from jax.experimental import pallas as pl
from jax.experimental.pallas import tpu as pltpu
from jax.experimental.pallas import tpu_sc as plsc

D = 128
NUM_CRYSTALS = 1000
CP = 1024
NP_ATOMS = 10240
CLAMP = 60.0
NC, NS, L = 2, 16, 16
NW = NC * NS


def _full(a):
    return pl.BlockSpec(a.shape, lambda i: (0,) * a.ndim)


def _row2(x):
    return x.reshape(1, -1)


def _tc_embed(orig, emb):
    N = orig.shape[0]
    W, b = emb
    B = 2000

    def body(x, w, br, out):
        out[...] = jnp.dot(x[...], w[...], preferred_element_type=jnp.float32) + br[...]

    return pl.pallas_call(
        body,
        grid=(N // B,),
        in_specs=[pl.BlockSpec((B, D), lambda i: (i, 0)), _full(W), _full(_row2(b))],
        out_specs=pl.BlockSpec((B, D), lambda i: (i, 0)),
        out_shape=jax.ShapeDtypeStruct((N, D), jnp.float32),
    )(orig, W, _row2(b))


def _gate_coeff(w, gate):
    return w * jnp.exp(jnp.clip(gate, -CLAMP, CLAMP))


def _tc_edge_mlp(self_f, nbr_f, nbr_w, msg, pool, block=1280):
    M = self_f.shape[0]
    (W1, b1), (W2, b2) = msg
    (P1, q1), (P2, q2), (P3, q3) = pool
    W1a, W1b = W1[:D], W1[D:]

    bf = jnp.bfloat16

    def body(sf, nf, nw, w1a, w1b, b1r, w2, b2r, p1, q1r, p2, q2r, p3, q3r,
             scaled_o, c_o):
        h = jnp.dot(sf[...].astype(bf), w1a[...],
                    preferred_element_type=jnp.float32)
        h += jnp.dot(nf[...].astype(bf), w1b[...],
                     preferred_element_type=jnp.float32)
        h = jnp.maximum(h + b1r[...], 0.0)
        fea = jnp.dot(h.astype(bf), w2[...],
                      preferred_element_type=jnp.float32) + b2r[...]
        g = jnp.maximum(
            jnp.dot(fea.astype(bf), p1[...],
                    preferred_element_type=jnp.float32) + q1r[...], 0.0)
        g = jnp.maximum(
            jnp.dot(g.astype(bf), p2[...],
                    preferred_element_type=jnp.float32) + q2r[...], 0.0)
        gate = jnp.dot(g, p3[...], preferred_element_type=jnp.float32) + q3r[...]
        c = _gate_coeff(nw[...], gate)
        scaled_o[...] = c * fea
        c_o[...] = c

    args = (self_f, nbr_f, nbr_w, W1a.astype(bf), W1b.astype(bf), _row2(b1),
            W2.astype(bf), _row2(b2), P1.astype(bf), _row2(q1),
            P2.astype(bf), _row2(q2), P3, _row2(q3))
    eb = pl.BlockSpec((block, D), lambda i: (i, 0))
    wb = pl.BlockSpec((block, 1), lambda i: (i, 0))
    return pl.pallas_call(
        body,
        grid=(M // block,),
        in_specs=[eb, eb, wb] + [_full(a) for a in args[3:]],
        out_specs=[eb, wb],
        out_shape=[jax.ShapeDtypeStruct((M, D), jnp.float32),
                   jax.ShapeDtypeStruct((M, 1), jnp.float32)],
    )(*args)


def _tc_combine(fea_partials, den_t, table):
    N = table.shape[0]
    B = 2000

    def body(p, dn, t, out):
        s = p[0] + p[1]
        den = jnp.sum(dn[...], axis=1, keepdims=True)
        out[...] = t[...] + s / (den + 1e-13)

    return pl.pallas_call(
        body,
        grid=(N // B,),
        in_specs=[pl.BlockSpec((NC, B, D), lambda i: (0, i, 0)),
                  pl.BlockSpec((B, NW), lambda i: (i, 0)),
                  pl.BlockSpec((B, D), lambda i: (i, 0))],
        out_specs=pl.BlockSpec((B, D), lambda i: (i, 0)),
        out_shape=jax.ShapeDtypeStruct((N, D), jnp.float32),
    )(fea_partials, den_t, table)


def _tc_cry_gate(table_p, weights_p, cry):
    NPad = table_p.shape[0]
    (C1, d1), (C2, d2), (C3, d3), (C4, d4) = cry
    B = 2048

    def body(t, w, c1, e1, c2, e2, c3, e3, c4, e4, scaled_o, c_o):
        x = t[...]
        h = jnp.maximum(jnp.dot(x, c1[...], preferred_element_type=jnp.float32) + e1[...], 0.0)
        h = jnp.maximum(jnp.dot(h, c2[...], preferred_element_type=jnp.float32) + e2[...], 0.0)
        h = jnp.maximum(jnp.dot(h, c3[...], preferred_element_type=jnp.float32) + e3[...], 0.0)
        gate = jnp.dot(h, c4[...], preferred_element_type=jnp.float32) + e4[...]
        c = _gate_coeff(w[...], gate)
        scaled_o[...] = c * x
        c_o[...] = c

    args = (table_p, weights_p, C1, _row2(d1), C2, _row2(d2), C3, _row2(d3),
            C4, _row2(d4))
    return pl.pallas_call(
        body,
        grid=(NPad // B,),
        in_specs=[pl.BlockSpec((B, D), lambda i: (i, 0)),
                  pl.BlockSpec((B, 1), lambda i: (i, 0))] +
                 [_full(a) for a in args[2:]],
        out_specs=[pl.BlockSpec((B, D), lambda i: (i, 0)),
                   pl.BlockSpec((B, 1), lambda i: (i, 0))],
        out_shape=[jax.ShapeDtypeStruct((NPad, D), jnp.float32),
                   jax.ShapeDtypeStruct((NPad, 1), jnp.float32)],
    )(*args)


def _tc_out(fea_partials, den_t, out_fcs, out_res, out_lin):
    Wo, bo = out_lin

    def body(p, dn, *refs):
        s = p[0] + p[1]
        den = jnp.sum(dn[...], axis=1, keepdims=True)
        fea = s / (den + 1e-13)
        args = refs[:-1]
        out = refs[-1]
        k = 0
        for _ in range(4):
            wf, bf, wr = args[k], args[k + 1], args[k + 2]
            k += 3
            fea = jnp.maximum(
                jnp.dot(fea, wf[...], preferred_element_type=jnp.float32) + bf[...], 0.0
            ) + jnp.dot(fea, wr[...], preferred_element_type=jnp.float32)
        out[...] = jnp.dot(fea, args[k][...], preferred_element_type=jnp.float32) + args[k + 1][...]

    flat = []
    for (Wf, bf), (Wr,) in zip(out_fcs, out_res):
        flat += [Wf, _row2(bf), Wr]
    flat += [Wo, _row2(bo)]
    return pl.pallas_call(
        body,
        grid=(1,),
        in_specs=[_full(fea_partials), _full(den_t)] + [_full(a) for a in flat],
        out_specs=pl.BlockSpec((CP, 2), lambda i: (0, 0)),
        out_shape=jax.ShapeDtypeStruct((CP, 2), jnp.float32),
    )(fea_partials, den_t, *flat)


def _sc_gather(table, weights, sidx3, nidx3, nb=4):
    N = table.shape[0]
    _, K, CH = sidx3.shape
    epw = K * CH
    M = NW * epw
    ko = K // nb
    with_w = weights is not None
    mesh = plsc.VectorSubcoreMesh(core_axis_name="c", subcore_axis_name="s")

    out_type = [jax.ShapeDtypeStruct((M, D), jnp.float32),
                jax.ShapeDtypeStruct((M, D), jnp.float32)]
    if with_w:
        out_type.append(jax.ShapeDtypeStruct((M,), jnp.float32))
    scratch = ([pltpu.VMEM((K, CH), jnp.int32)] * 2 +
               ([pltpu.VMEM((N,), jnp.float32)] if with_w else []) +
               [pltpu.VMEM((CH, D), jnp.float32)] * (2 * nb) +
               ([pltpu.VMEM((CH,), jnp.float32)] * nb if with_w else []) +
               [pltpu.SemaphoreType.DMA] * (3 * nb))

    @functools.partial(
        pl.kernel,
        out_type=out_type,
        mesh=mesh,
        scratch_types=scratch,
        compiler_params=pltpu.CompilerParams(needs_layout_passes=False),
    )
    def k(*refs):
        if with_w:
            (table_hbm, w_hbm, sidx_hbm, nidx_hbm,
             self_out, nbr_out, w_out), scr = refs[:7], refs[7:]
        else:
            (table_hbm, sidx_hbm, nidx_hbm,
             self_out, nbr_out), scr = refs[:5], refs[5:]
        sidxv, nidxv = scr[0], scr[1]
        scr = scr[2:]
        if with_w:
            wtab = scr[0]
            scr = scr[1:]
        srv = scr[:nb]
        nrv = scr[nb:2 * nb]
        scr = scr[2 * nb:]
        if with_w:
            wv = scr[:nb]
            scr = scr[nb:]
        sg = scr[:nb]
        ng = scr[nb:2 * nb]
        ws = scr[2 * nb:3 * nb]
        wid = lax.axis_index("s") * NC + lax.axis_index("c")
        if with_w:
            pltpu.sync_copy(w_hbm, wtab)
        pltpu.sync_copy(sidx_hbm.at[wid], sidxv)
        pltpu.sync_copy(nidx_hbm.at[wid], nidxv)

        def chunk_gather(c, b):
            return (pltpu.async_copy(table_hbm.at[sidxv.at[c]], srv[b], sg[b]),
                    pltpu.async_copy(table_hbm.at[nidxv.at[c]], nrv[b], ng[b]))

        def chunk_drain(c, b, h1, h2):
            base = wid * epw + c * CH
            if with_w:
                for j in range(CH // L):
                    sl = pl.ds(j * L, L)
                    wv[b][sl] = plsc.load_gather(wtab, [nidxv[c, sl]])
            h1.wait()
            h2.wait()
            out = [pltpu.async_copy(srv[b], self_out.at[pl.ds(base, CH)], ws[b]),
                   pltpu.async_copy(nrv[b], nbr_out.at[pl.ds(base, CH)], ws[b])]
            if with_w:
                out.append(
                    pltpu.async_copy(wv[b], w_out.at[pl.ds(base, CH)], ws[b]))
            return out

        def step(kk, carry):
            hs = [chunk_gather(kk * nb + b, b) for b in range(nb)]
            whs = []
            for b in range(nb):
                whs += chunk_drain(kk * nb + b, b, *hs[b])
            for h in whs:
                h.wait()
            return carry

        lax.fori_loop(0, ko, step, 0)
        whs = []
        hs = [chunk_gather(wid * 0 + c, c - ko * nb) for c in range(ko * nb, K)]
        for c in range(ko * nb, K):
            whs += chunk_drain(wid * 0 + c, c - ko * nb, *hs[c - ko * nb])
        for h in whs:
            h.wait()

    if with_w:
        return k(table, weights, sidx3, nidx3)
    return k(table, sidx3, nidx3)


def _sc_segment_sum(scaled, cvec, idx3, zeros_seg):
    _, K, CH = idx3.shape
    epw = K * CH
    nseg = zeros_seg.shape[0]
    mesh = plsc.VectorSubcoreMesh(core_axis_name="c", subcore_axis_name="s")

    nb = 1 if nseg * D > 600_000 else min(2, K)
    ko = K // nb
    rpt = nseg // NS - (nseg // NS) % 8
    tail_rows = nseg - rpt * NS
    scratch = ([pltpu.VMEM((nseg,), jnp.float32),
                pltpu.VMEM_SHARED((nseg, D), jnp.float32),
                pltpu.VMEM((K, CH), jnp.int32),
                pltpu.VMEM((epw,), jnp.float32)] +
               [pltpu.VMEM((CH, D), jnp.float32)] * nb +
               [pltpu.SemaphoreType.DMA] * (2 * nb))

    @functools.partial(
        pl.kernel,
        out_type=[jax.ShapeDtypeStruct((NC, nseg, D), jnp.float32),
                  jax.ShapeDtypeStruct((NW, nseg), jnp.float32)],
        mesh=mesh,
        scratch_types=scratch,
        compiler_params=pltpu.CompilerParams(needs_layout_passes=False),
    )
    def k(scaled_hbm, cvec_hbm, idx_hbm, zeros_hbm, fea_out, den_out, *scr):
        dacc, acc, idxv, cvecv = scr[0], scr[1], scr[2], scr[3]
        rv = scr[4:4 + nb]
        sl_sem = scr[4 + nb:4 + 2 * nb]
        ss_sem = scr[4 + 2 * nb:4 + 3 * nb]
        cid = lax.axis_index("c")
        sid = lax.axis_index("s")
        wid = sid * NC + cid

        @pl.when(sid == 0)
        def _init():
            pltpu.sync_copy(zeros_hbm, acc)

        pltpu.sync_copy(idx_hbm.at[wid], idxv)
        pltpu.sync_copy(cvec_hbm.at[pl.ds(wid * epw, epw)], cvecv)
        zeros16 = jnp.zeros((L,), jnp.float32)

        def zstep(j, carry):
            dacc[pl.ds(j * L, L)] = zeros16
            return carry

        lax.fori_loop(0, nseg // L, zstep, 0)
        plsc.subcore_barrier()

        def chunk_load(c, b):
            base = wid * epw + c * CH
            return pltpu.async_copy(scaled_hbm.at[pl.ds(base, CH)], rv[b],
                                    sl_sem[b])

        def chunk_add(c, b, h):
            h.wait()
            out = pltpu.async_copy(rv[b], acc.at[idxv.at[c]], ss_sem[b],
                                   add=True)
            for j in range(CH // L):
                plsc.addupdate_scatter(dacc, [idxv[c, pl.ds(j * L, L)]],
                                       cvecv[pl.ds(c * CH + j * L, L)])
            return out

        def step(kk, carry):
            hs = [chunk_load(kk * nb + b, b) for b in range(nb)]
            shs = [chunk_add(kk * nb + b, b, hs[b]) for b in range(nb)]
            for h in shs:
                h.wait()
            return carry

        lax.fori_loop(0, ko, step, 0)
        hs = [chunk_load(wid * 0 + c, c - ko * nb) for c in range(ko * nb, K)]
        shs = [chunk_add(wid * 0 + c, c - ko * nb, hs[c - ko * nb])
               for c in range(ko * nb, K)]
        for h in shs:
            h.wait()
        plsc.subcore_barrier()
        r0 = sid * rpt
        pltpu.sync_copy(acc.at[pl.ds(r0, rpt)],
                        fea_out.at[cid, pl.ds(r0, rpt)])
        if tail_rows:
            @pl.when(sid == 0)
            def _dump_tail():
                pltpu.sync_copy(
                    acc.at[pl.ds(rpt * NS, tail_rows)],
                    fea_out.at[cid, pl.ds(rpt * NS, tail_rows)])
        pltpu.sync_copy(dacc, den_out.at[wid])

    return k(scaled, cvec, idx3, zeros_seg)


def kernel(atom_weights, orig_atom_fea, params, self_fea_idx, nbr_fea_idx,
           crystal_atom_idx):
    aw = atom_weights.astype(jnp.float32)
    sidx = self_fea_idx.astype(jnp.int32)
    nidx = nbr_fea_idx.astype(jnp.int32)
    cidx = crystal_atom_idx.astype(jnp.int32)
    N = aw.shape[0]
    M = sidx.shape[0]
    aw1 = aw.reshape(N)

    che = 80
    ke = M // (NW * che)
    sidx3 = sidx.reshape(NW, ke, che)
    nidx3 = nidx.reshape(NW, ke, che)
    zerosN = jnp.zeros((N, D), jnp.float32)

    table = _tc_embed(orig_atom_fea, params["emb"])
    nbr_w2 = None
    for g in params["graphs"]:
        if nbr_w2 is None:
            self_f, nbr_f, nbr_w = _sc_gather(table, aw1, sidx3, nidx3)
            nbr_w2 = nbr_w.reshape(M, 1)
        else:
            self_f, nbr_f = _sc_gather(table, None, sidx3, nidx3)
        scaled, cvec = _tc_edge_mlp(self_f, nbr_f, nbr_w2, g["msg"], g["pool"])
        fea_p, den_p = _sc_segment_sum(scaled, cvec.reshape(M), sidx3, zerosN)
        table = _tc_combine(fea_p, den_p.T, table)

    table_p = jnp.pad(table, ((0, NP_ATOMS - N), (0, 0)))
    aw_p = jnp.pad(aw, ((0, NP_ATOMS - N), (0, 0)))
    scaled_c, cvec_c = _tc_cry_gate(table_p, aw_p, params["cry"])
    chc = 64
    kc = NP_ATOMS // (NW * chc)
    cidx3 = jnp.pad(cidx, (0, NP_ATOMS - N)).reshape(NW, kc, chc)
    zerosC = jnp.zeros((CP, D), jnp.float32)
    fea_pc, den_pc = _sc_segment_sum(scaled_c, cvec_c.reshape(NP_ATOMS),
                                     cidx3, zerosC)
    out = _tc_out(fea_pc, den_pc.T, params["out_fcs"], params["out_res"],
                  params["out"])
    return out[:NUM_CRYSTALS]


# speedup vs baseline: 1.1077x; 1.0464x over previous
import functools

import jax
import jax.numpy as jnp
from jax import lax
from jax.experimental import pallas as pl
from jax.experimental.pallas import tpu as pltpu
from jax.experimental.pallas import tpu_sc as plsc

D = 128
NUM_CRYSTALS = 1000
CP = 1024
NP_ATOMS = 10240
CLAMP = 60.0
NC, NS, L = 2, 16, 16
NW = NC * NS


def _full(a):
    return pl.BlockSpec(a.shape, lambda i: (0,) * a.ndim)


def _row2(x):
    return x.reshape(1, -1)


def _tc_embed(orig, emb):
    N = orig.shape[0]
    W, b = emb
    B = 2000

    def body(x, w, br, out):
        out[...] = jnp.dot(x[...], w[...], preferred_element_type=jnp.float32) + br[...]

    return pl.pallas_call(
        body,
        grid=(N // B,),
        in_specs=[pl.BlockSpec((B, D), lambda i: (i, 0)), _full(W), _full(_row2(b))],
        out_specs=pl.BlockSpec((B, D), lambda i: (i, 0)),
        out_shape=jax.ShapeDtypeStruct((N, D), jnp.float32),
    )(orig, W, _row2(b))


def _gate_coeff(w, gate):
    return w * jnp.exp(jnp.clip(gate, -CLAMP, CLAMP))


def _tc_edge_mlp(self_f, nbr_f, nbr_w, msg, pool, block=1280):
    M = self_f.shape[0]
    (W1, b1), (W2, b2) = msg
    (P1, q1), (P2, q2), (P3, q3) = pool
    W1a, W1b = W1[:D], W1[D:]

    bf = jnp.bfloat16

    def body(sf, nf, nw, w1a, w1b, b1r, w2, b2r, p1, q1r, p2, q2r, p3, q3r,
             scaled_o, c_o):
        h = jnp.dot(sf[...].astype(bf), w1a[...],
                    preferred_element_type=jnp.float32)
        h += jnp.dot(nf[...].astype(bf), w1b[...],
                     preferred_element_type=jnp.float32)
        h = jnp.maximum(h + b1r[...], 0.0)
        fea = jnp.dot(h.astype(bf), w2[...],
                      preferred_element_type=jnp.float32) + b2r[...]
        g = jnp.maximum(
            jnp.dot(fea.astype(bf), p1[...],
                    preferred_element_type=jnp.float32) + q1r[...], 0.0)
        g = jnp.maximum(
            jnp.dot(g.astype(bf), p2[...],
                    preferred_element_type=jnp.float32) + q2r[...], 0.0)
        gate = jnp.dot(g, p3[...], preferred_element_type=jnp.float32) + q3r[...]
        c = _gate_coeff(nw[...], gate)
        scaled_o[...] = c * fea
        c_o[...] = c

    args = (self_f, nbr_f, nbr_w, W1a.astype(bf), W1b.astype(bf), _row2(b1),
            W2.astype(bf), _row2(b2), P1.astype(bf), _row2(q1),
            P2.astype(bf), _row2(q2), P3, _row2(q3))
    eb = pl.BlockSpec((block, D), lambda i: (i, 0))
    wb = pl.BlockSpec((block, 1), lambda i: (i, 0))
    return pl.pallas_call(
        body,
        grid=(M // block,),
        in_specs=[eb, eb, wb] + [_full(a) for a in args[3:]],
        out_specs=[eb, wb],
        out_shape=[jax.ShapeDtypeStruct((M, D), jnp.float32),
                   jax.ShapeDtypeStruct((M, 1), jnp.float32)],
    )(*args)


def _tc_combine(fea_partials, den_t, table):
    N = table.shape[0]
    B = 2000

    def body(p, dn, t, out):
        s = p[0] + p[1]
        den = jnp.sum(dn[...], axis=1, keepdims=True)
        out[...] = t[...] + s / (den + 1e-13)

    return pl.pallas_call(
        body,
        grid=(N // B,),
        in_specs=[pl.BlockSpec((NC, B, D), lambda i: (0, i, 0)),
                  pl.BlockSpec((B, NW), lambda i: (i, 0)),
                  pl.BlockSpec((B, D), lambda i: (i, 0))],
        out_specs=pl.BlockSpec((B, D), lambda i: (i, 0)),
        out_shape=jax.ShapeDtypeStruct((N, D), jnp.float32),
    )(fea_partials, den_t, table)


def _tc_cry_gate(table_p, weights_p, cry):
    NPad = table_p.shape[0]
    (C1, d1), (C2, d2), (C3, d3), (C4, d4) = cry
    B = 2048

    def body(t, w, c1, e1, c2, e2, c3, e3, c4, e4, scaled_o, c_o):
        x = t[...]
        h = jnp.maximum(jnp.dot(x, c1[...], preferred_element_type=jnp.float32) + e1[...], 0.0)
        h = jnp.maximum(jnp.dot(h, c2[...], preferred_element_type=jnp.float32) + e2[...], 0.0)
        h = jnp.maximum(jnp.dot(h, c3[...], preferred_element_type=jnp.float32) + e3[...], 0.0)
        gate = jnp.dot(h, c4[...], preferred_element_type=jnp.float32) + e4[...]
        c = _gate_coeff(w[...], gate)
        scaled_o[...] = c * x
        c_o[...] = c

    args = (table_p, weights_p, C1, _row2(d1), C2, _row2(d2), C3, _row2(d3),
            C4, _row2(d4))
    return pl.pallas_call(
        body,
        grid=(NPad // B,),
        in_specs=[pl.BlockSpec((B, D), lambda i: (i, 0)),
                  pl.BlockSpec((B, 1), lambda i: (i, 0))] +
                 [_full(a) for a in args[2:]],
        out_specs=[pl.BlockSpec((B, D), lambda i: (i, 0)),
                   pl.BlockSpec((B, 1), lambda i: (i, 0))],
        out_shape=[jax.ShapeDtypeStruct((NPad, D), jnp.float32),
                   jax.ShapeDtypeStruct((NPad, 1), jnp.float32)],
    )(*args)


def _tc_out(fea_partials, den_t, out_fcs, out_res, out_lin):
    Wo, bo = out_lin

    def body(p, dn, *refs):
        s = p[0] + p[1]
        den = jnp.sum(dn[...], axis=1, keepdims=True)
        fea = s / (den + 1e-13)
        args = refs[:-1]
        out = refs[-1]
        k = 0
        for _ in range(4):
            wf, bf, wr = args[k], args[k + 1], args[k + 2]
            k += 3
            fea = jnp.maximum(
                jnp.dot(fea, wf[...], preferred_element_type=jnp.float32) + bf[...], 0.0
            ) + jnp.dot(fea, wr[...], preferred_element_type=jnp.float32)
        out[...] = jnp.dot(fea, args[k][...], preferred_element_type=jnp.float32) + args[k + 1][...]

    flat = []
    for (Wf, bf), (Wr,) in zip(out_fcs, out_res):
        flat += [Wf, _row2(bf), Wr]
    flat += [Wo, _row2(bo)]
    return pl.pallas_call(
        body,
        grid=(1,),
        in_specs=[_full(fea_partials), _full(den_t)] + [_full(a) for a in flat],
        out_specs=pl.BlockSpec((CP, 2), lambda i: (0, 0)),
        out_shape=jax.ShapeDtypeStruct((CP, 2), jnp.float32),
    )(fea_partials, den_t, *flat)


def _sc_gather(table, weights, sidx3, nidx3, nb=4):
    N = table.shape[0]
    _, K, CH = sidx3.shape
    epw = K * CH
    M = NW * epw
    ko = K // nb
    with_w = weights is not None
    mesh = plsc.VectorSubcoreMesh(core_axis_name="c", subcore_axis_name="s")

    out_type = [jax.ShapeDtypeStruct((M, D), jnp.float32),
                jax.ShapeDtypeStruct((M, D), jnp.float32)]
    if with_w:
        out_type.append(jax.ShapeDtypeStruct((M,), jnp.float32))
    scratch = ([pltpu.VMEM((K, CH), jnp.int32)] * 2 +
               ([pltpu.VMEM((N,), jnp.float32)] if with_w else []) +
               [pltpu.VMEM((CH, D), jnp.float32)] * (2 * nb) +
               ([pltpu.VMEM((CH,), jnp.float32)] * nb if with_w else []) +
               [pltpu.SemaphoreType.DMA] * (3 * nb))

    @functools.partial(
        pl.kernel,
        out_type=out_type,
        mesh=mesh,
        scratch_types=scratch,
        compiler_params=pltpu.CompilerParams(needs_layout_passes=False),
    )
    def k(*refs):
        if with_w:
            (table_hbm, w_hbm, sidx_hbm, nidx_hbm,
             self_out, nbr_out, w_out), scr = refs[:7], refs[7:]
        else:
            (table_hbm, sidx_hbm, nidx_hbm,
             self_out, nbr_out), scr = refs[:5], refs[5:]
        sidxv, nidxv = scr[0], scr[1]
        scr = scr[2:]
        if with_w:
            wtab = scr[0]
            scr = scr[1:]
        srv = scr[:nb]
        nrv = scr[nb:2 * nb]
        scr = scr[2 * nb:]
        if with_w:
            wv = scr[:nb]
            scr = scr[nb:]
        sg = scr[:nb]
        ng = scr[nb:2 * nb]
        ws = scr[2 * nb:3 * nb]
        wid = lax.axis_index("s") * NC + lax.axis_index("c")
        if with_w:
            pltpu.sync_copy(w_hbm, wtab)
        pltpu.sync_copy(sidx_hbm.at[wid], sidxv)
        pltpu.sync_copy(nidx_hbm.at[wid], nidxv)

        def chunk_gather(c, b):
            return (pltpu.async_copy(table_hbm.at[sidxv.at[c]], srv[b], sg[b]),
                    pltpu.async_copy(table_hbm.at[nidxv.at[c]], nrv[b], ng[b]))

        def chunk_drain(c, b, h1, h2):
            base = wid * epw + c * CH
            if with_w:
                for j in range(CH // L):
                    sl = pl.ds(j * L, L)
                    wv[b][sl] = plsc.load_gather(wtab, [nidxv[c, sl]])
            h1.wait()
            h2.wait()
            out = [pltpu.async_copy(srv[b], self_out.at[pl.ds(base, CH)], ws[b]),
                   pltpu.async_copy(nrv[b], nbr_out.at[pl.ds(base, CH)], ws[b])]
            if with_w:
                out.append(
                    pltpu.async_copy(wv[b], w_out.at[pl.ds(base, CH)], ws[b]))
            return out

        def step(kk, carry):
            hs = [chunk_gather(kk * nb + b, b) for b in range(nb)]
            whs = []
            for b in range(nb):
                whs += chunk_drain(kk * nb + b, b, *hs[b])
            for h in whs:
                h.wait()
            return carry

        lax.fori_loop(0, ko, step, 0)
        whs = []
        hs = [chunk_gather(wid * 0 + c, c - ko * nb) for c in range(ko * nb, K)]
        for c in range(ko * nb, K):
            whs += chunk_drain(wid * 0 + c, c - ko * nb, *hs[c - ko * nb])
        for h in whs:
            h.wait()

    if with_w:
        return k(table, weights, sidx3, nidx3)
    return k(table, sidx3, nidx3)


def _sc_segment_sum(scaled, cvec, idx3, zeros_seg):
    _, K, CH = idx3.shape
    epw = K * CH
    nseg = zeros_seg.shape[0]
    mesh = plsc.VectorSubcoreMesh(core_axis_name="c", subcore_axis_name="s")

    nb = min(2, K)
    ko = K // nb
    rpt = nseg // NS - (nseg // NS) % 8
    tail_rows = nseg - rpt * NS
    scratch = ([pltpu.VMEM((nseg,), jnp.float32),
                pltpu.VMEM_SHARED((nseg, D), jnp.float32),
                pltpu.VMEM((K, CH), jnp.int32)] +
               [pltpu.VMEM((CH,), jnp.float32)] * nb +
               [pltpu.VMEM((CH, D), jnp.float32)] * nb +
               [pltpu.SemaphoreType.DMA] * (2 * nb))

    @functools.partial(
        pl.kernel,
        out_type=[jax.ShapeDtypeStruct((NC, nseg, D), jnp.float32),
                  jax.ShapeDtypeStruct((NW, nseg), jnp.float32)],
        mesh=mesh,
        scratch_types=scratch,
        compiler_params=pltpu.CompilerParams(needs_layout_passes=False),
    )
    def k(scaled_hbm, cvec_hbm, idx_hbm, zeros_hbm, fea_out, den_out, *scr):
        dacc, acc, idxv = scr[0], scr[1], scr[2]
        cv = scr[3:3 + nb]
        rv = scr[3 + nb:3 + 2 * nb]
        sl_sem = scr[3 + 2 * nb:3 + 3 * nb]
        ss_sem = scr[3 + 3 * nb:3 + 4 * nb]
        cid = lax.axis_index("c")
        sid = lax.axis_index("s")
        wid = sid * NC + cid

        @pl.when(sid == 0)
        def _init():
            pltpu.sync_copy(zeros_hbm, acc)

        pltpu.sync_copy(idx_hbm.at[wid], idxv)
        zeros16 = jnp.zeros((L,), jnp.float32)

        def zstep(j, carry):
            dacc[pl.ds(j * L, L)] = zeros16
            return carry

        lax.fori_loop(0, nseg // L, zstep, 0)
        plsc.subcore_barrier()

        def chunk_load(c, b):
            base = wid * epw + c * CH
            h = pltpu.async_copy(scaled_hbm.at[pl.ds(base, CH)], rv[b],
                                 sl_sem[b])
            pltpu.sync_copy(cvec_hbm.at[pl.ds(base, CH)], cv[b])
            return h

        def chunk_add(c, b, h):
            h.wait()
            out = pltpu.async_copy(rv[b], acc.at[idxv.at[c]], ss_sem[b],
                                   add=True)
            for j in range(CH // L):
                plsc.addupdate_scatter(dacc, [idxv[c, pl.ds(j * L, L)]],
                                       cv[b][pl.ds(j * L, L)])
            return out

        def step(kk, carry):
            hs = [chunk_load(kk * nb + b, b) for b in range(nb)]
            shs = [chunk_add(kk * nb + b, b, hs[b]) for b in range(nb)]
            for h in shs:
                h.wait()
            return carry

        lax.fori_loop(0, ko, step, 0)
        hs = [chunk_load(wid * 0 + c, c - ko * nb) for c in range(ko * nb, K)]
        shs = [chunk_add(wid * 0 + c, c - ko * nb, hs[c - ko * nb])
               for c in range(ko * nb, K)]
        for h in shs:
            h.wait()
        plsc.subcore_barrier()
        r0 = sid * rpt
        pltpu.sync_copy(acc.at[pl.ds(r0, rpt)],
                        fea_out.at[cid, pl.ds(r0, rpt)])
        if tail_rows:
            @pl.when(sid == 0)
            def _dump_tail():
                pltpu.sync_copy(
                    acc.at[pl.ds(rpt * NS, tail_rows)],
                    fea_out.at[cid, pl.ds(rpt * NS, tail_rows)])
        pltpu.sync_copy(dacc, den_out.at[wid])

    return k(scaled, cvec, idx3, zeros_seg)


def kernel(atom_weights, orig_atom_fea, params, self_fea_idx, nbr_fea_idx,
           crystal_atom_idx):
    aw = atom_weights.astype(jnp.float32)
    sidx = self_fea_idx.astype(jnp.int32)
    nidx = nbr_fea_idx.astype(jnp.int32)
    cidx = crystal_atom_idx.astype(jnp.int32)
    N = aw.shape[0]
    M = sidx.shape[0]
    aw1 = aw.reshape(N)

    che = 80
    ke = M // (NW * che)
    sidx3 = sidx.reshape(NW, ke, che)
    nidx3 = nidx.reshape(NW, ke, che)
    zerosN = jnp.zeros((N, D), jnp.float32)

    table = _tc_embed(orig_atom_fea, params["emb"])
    nbr_w2 = None
    for g in params["graphs"]:
        if nbr_w2 is None:
            self_f, nbr_f, nbr_w = _sc_gather(table, aw1, sidx3, nidx3)
            nbr_w2 = nbr_w.reshape(M, 1)
        else:
            self_f, nbr_f = _sc_gather(table, None, sidx3, nidx3)
        scaled, cvec = _tc_edge_mlp(self_f, nbr_f, nbr_w2, g["msg"], g["pool"])
        fea_p, den_p = _sc_segment_sum(scaled, cvec.reshape(M), sidx3, zerosN)
        table = _tc_combine(fea_p, den_p.T, table)

    table_p = jnp.pad(table, ((0, NP_ATOMS - N), (0, 0)))
    aw_p = jnp.pad(aw, ((0, NP_ATOMS - N), (0, 0)))
    scaled_c, cvec_c = _tc_cry_gate(table_p, aw_p, params["cry"])
    chc = 64
    kc = NP_ATOMS // (NW * chc)
    cidx3 = jnp.pad(cidx, (0, NP_ATOMS - N)).reshape(NW, kc, chc)
    zerosC = jnp.zeros((CP, D), jnp.float32)
    fea_pc, den_pc = _sc_segment_sum(scaled_c, cvec_c.reshape(NP_ATOMS),
                                     cidx3, zerosC)
    out = _tc_out(fea_pc, den_pc.T, params["out_fcs"], params["out_res"],
                  params["out"])
    return out[:NUM_CRYSTALS]
